# baseline (speedup 1.0000x reference)
import jax
import jax.numpy as jnp
from jax.experimental import pallas as pl
from jax.experimental.pallas import tpu as pltpu

L = 32
B = 256
D = 128
NEG = -1e30


def _softmax_exp(x):
    e = jnp.exp(x - jnp.max(x, axis=-1, keepdims=True))
    return e / jnp.sum(e, axis=-1, keepdims=True)


def _kldiv_parts(p, q):
    p = p + 1e-9
    q = q + 1e-9
    sp = jnp.sum(p, axis=-1, keepdims=True)
    sq = jnp.sum(q, axis=-1, keepdims=True)
    t = jnp.sum(p * jnp.log(p / q), axis=-1, keepdims=True)
    return t, sp, sq


def _kldiv3(p, q):
    t, sp, sq = _kldiv_parts(p, q)
    return t / sp + jnp.log(sq / sp)


def _autoencode(cat2, w_in, b_in, w_out, b_out):
    h = jnp.dot(cat2, w_in, preferred_element_type=jnp.float32) + b_in
    vecm = _softmax_exp(h)
    rec = jnp.dot(vecm, w_out, preferred_element_type=jnp.float32) + b_out
    recl = _softmax_exp(rec[:, :D])
    recr = _softmax_exp(rec[:, D:])
    return vecm, recl, recr


def _rae_body(x_in_ref, w_in_ref, b_in_ref, w_out_ref, b_out_ref,
              out_ref, x_ref, vm_ref, nxt_ref, prv_ref, perp_ref):
    w_in = w_in_ref[...]
    b_in = b_in_ref[...]
    w_out = w_out_ref[...]
    b_out = b_out_ref[...]

    iota_pl = jax.lax.broadcasted_iota(jnp.int32, (L - 1, 1, B), 0)
    iota_ll = jax.lax.broadcasted_iota(jnp.int32, (L, 1, B), 0)
    iota_pf = jax.lax.broadcasted_iota(
        jnp.int32, (L - 1, B, 1), 0).astype(jnp.float32)
    iota_lf = jax.lax.broadcasted_iota(
        jnp.int32, (L, B, 1), 0).astype(jnp.float32)

    x = x_in_ref[...]
    x_ref[...] = x
    vecl = x[: L - 1]
    vecr = x[1:L]
    cat2 = jnp.concatenate([vecl, vecr], axis=-1).reshape((L - 1) * B, 2 * D)
    vecm, recl, recr = _autoencode(cat2, w_in, b_in, w_out, b_out)
    vm_ref[...] = vecm.reshape(L - 1, B, D)
    tl, spl, sql = _kldiv_parts(recl.reshape(L - 1, B, D), vecl)
    tr, spr, sqr = _kldiv_parts(recr.reshape(L - 1, B, D), vecr)
    parts = jnp.concatenate([tl, spl, sql, tr, spr, sqr], axis=0)
    pT = jnp.swapaxes(parts, 1, 2).reshape(6, L - 1, 1, B)
    perp_ref[...] = (pT[0] / pT[1] + jnp.log(pT[2] / pT[1])
                     + pT[3] / pT[4] + jnp.log(pT[5] / pT[4]))
    nxt_ref[...] = iota_ll + 1
    prv_ref[...] = iota_ll - 1

    def step(t, tot):
        perp = perp_ref[...]
        maxv = jnp.max(perp, axis=0, keepdims=True)
        a = jnp.min(jnp.where(perp == maxv, iota_pl, L - 1),
                    axis=0, keepdims=True)
        tot = tot + maxv

        nxt = nxt_ref[...]
        prv = prv_ref[...]
        oh_a = iota_ll == a
        j = jnp.sum(jnp.where(oh_a, nxt, 0), axis=0, keepdims=True)
        pp = jnp.sum(jnp.where(oh_a, prv, 0), axis=0, keepdims=True)
        nj = jnp.sum(jnp.where(iota_ll == j, nxt, 0), axis=0, keepdims=True)

        idxs = jnp.concatenate([a, pp, nj], axis=0).astype(jnp.float32)
        idxs_T = jnp.swapaxes(idxs, 1, 2)
        a_T = idxs_T[0:1]
        pp_T = idxs_T[1:2]
        nj_T = idxs_T[2:3]

        m_a = iota_lf == a_T
        m_pp = iota_lf == pp_T
        m_nj = iota_lf == nj_T

        x = x_ref[...]
        vecm_sel = jnp.sum(jnp.where(m_a[: L - 1], vm_ref[...], 0.0),
                           axis=0, keepdims=True)
        row_pp = jnp.sum(jnp.where(m_pp, x, 0.0),
                         axis=0, keepdims=True)
        row_nj = jnp.sum(jnp.where(m_nj, x, 0.0),
                         axis=0, keepdims=True)

        x_ref[...] = jnp.where(m_a, vecm_sel, x)
        nxt_ref[...] = jnp.where(oh_a, nj, nxt)
        prv_ref[...] = jnp.where(iota_ll == nj, a, prv)

        lrows = jnp.concatenate([row_pp, vecm_sel], axis=0)
        rrows = jnp.concatenate([vecm_sel, row_nj], axis=0)
        cat2 = jnp.concatenate([lrows, rrows], axis=-1).reshape(2 * B, 2 * D)
        vecm2, recl2, recr2 = _autoencode(cat2, w_in, b_in, w_out, b_out)
        vecm23 = vecm2.reshape(2, B, D)
        kl = (_kldiv3(recl2.reshape(2, B, D), lrows)
              + _kldiv3(recr2.reshape(2, B, D), rrows))
        kl_T = jnp.swapaxes(kl, 1, 2)

        perp = jnp.where(iota_pl == j, NEG, perp)
        perp = jnp.where(iota_pl == pp, kl_T[0:1], perp)
        perp = jnp.where(iota_pl == a,
                         jnp.where(nj < L, kl_T[1:2], NEG), perp)
        perp_ref[...] = perp
        vm = vm_ref[...]
        vm = jnp.where(m_pp[: L - 1], vecm23[0:1], vm)
        vm = jnp.where(m_a[: L - 1], vecm23[1:2], vm)
        vm_ref[...] = vm
        return tot

    tot = jax.lax.fori_loop(0, L - 1, step,
                            jnp.zeros((1, 1, B), jnp.float32))
    out_ref[...] = jnp.reshape(jnp.sum(tot) / 2.0 / B / L, (1, 1))


def kernel(input, W_in, b_in, W_outl, b_outl, W_outr, b_outr):
    w_out = jnp.concatenate([W_outl, W_outr], axis=1)
    b_out = jnp.concatenate([b_outl, b_outr])[None, :]
    b_in2 = b_in[None, :]
    out = pl.pallas_call(
        _rae_body,
        out_shape=jax.ShapeDtypeStruct((1, 1), jnp.float32),
        scratch_shapes=[
            pltpu.VMEM((L, B, D), jnp.float32),
            pltpu.VMEM((L - 1, B, D), jnp.float32),
            pltpu.VMEM((L, 1, B), jnp.int32),
            pltpu.VMEM((L, 1, B), jnp.int32),
            pltpu.VMEM((L - 1, 1, B), jnp.float32),
        ],
        compiler_params=pltpu.CompilerParams(
            vmem_limit_bytes=100 * 1024 * 1024),
    )(input, W_in, b_in2, w_out, b_out)
    return out.reshape(1)


# speedup vs baseline: 1.0516x; 1.0516x over previous
import jax
import jax.numpy as jnp
from jax.experimental import pallas as pl
from jax.experimental.pallas import tpu as pltpu

L = 32
B = 256
D = 128
NEG = -1e30


def _softmax_exp(x):
    e = jnp.exp(x - jnp.max(x, axis=-1, keepdims=True))
    return e / jnp.sum(e, axis=-1, keepdims=True)


def _kldiv_parts(p, q):
    p = p + 1e-9
    q = q + 1e-9
    sp = jnp.sum(p, axis=-1, keepdims=True)
    sq = jnp.sum(q, axis=-1, keepdims=True)
    t = jnp.sum(p * jnp.log(p / q), axis=-1, keepdims=True)
    return t, sp, sq


def _kldiv3(p, q):
    t, sp, sq = _kldiv_parts(p, q)
    return t / sp + jnp.log(sq / sp)


def _autoencode(cat2, w_in, b_in, w_out, b_out):
    h = jnp.dot(cat2, w_in, preferred_element_type=jnp.float32) + b_in
    vecm = _softmax_exp(h)
    rec = jnp.dot(vecm, w_out, preferred_element_type=jnp.float32) + b_out
    recl = _softmax_exp(rec[:, :D])
    recr = _softmax_exp(rec[:, D:])
    return vecm, recl, recr


def _rae_body(x_in_ref, w_in_ref, b_in_ref, w_out_ref, b_out_ref,
              out_ref, x_ref, vm_ref, nxt_ref, prv_ref, perp_ref):
    w_in = w_in_ref[...]
    b_in = b_in_ref[...]
    w_out = w_out_ref[...]
    b_out = b_out_ref[...]

    iota_pl = jax.lax.broadcasted_iota(jnp.int32, (L - 1, 1, B), 0)
    iota_ll = jax.lax.broadcasted_iota(jnp.int32, (L, 1, B), 0)
    iota_pf = jax.lax.broadcasted_iota(
        jnp.int32, (L - 1, B, 1), 0).astype(jnp.float32)
    iota_lf = jax.lax.broadcasted_iota(
        jnp.int32, (L, B, 1), 0).astype(jnp.float32)

    x = x_in_ref[...]
    x_ref[...] = x
    vecl = x[: L - 1]
    vecr = x[1:L]
    cat2 = jnp.concatenate([vecl, vecr], axis=-1).reshape((L - 1) * B, 2 * D)
    vecm, recl, recr = _autoencode(cat2, w_in, b_in, w_out, b_out)
    vm_ref[...] = vecm.reshape(L - 1, B, D)
    tl, spl, sql = _kldiv_parts(recl.reshape(L - 1, B, D), vecl)
    tr, spr, sqr = _kldiv_parts(recr.reshape(L - 1, B, D), vecr)
    parts = jnp.concatenate([tl, spl, sql, tr, spr, sqr], axis=0)
    pT = jnp.swapaxes(parts, 1, 2).reshape(6, L - 1, 1, B)
    perp_ref[...] = (pT[0] / pT[1] + jnp.log(pT[2] / pT[1])
                     + pT[3] / pT[4] + jnp.log(pT[5] / pT[4]))
    nxt_ref[...] = iota_ll + 1
    prv_ref[...] = iota_ll - 1

    def step(t, tot):
        perp = perp_ref[...]
        maxv = jnp.max(perp, axis=0, keepdims=True)
        a = jnp.min(jnp.where(perp == maxv, iota_pl, L - 1),
                    axis=0, keepdims=True)
        tot = tot + maxv

        nxt = nxt_ref[...]
        prv = prv_ref[...]
        oh_a = iota_ll == a
        j = jnp.sum(jnp.where(oh_a, nxt, 0), axis=0, keepdims=True)
        pp = jnp.sum(jnp.where(oh_a, prv, 0), axis=0, keepdims=True)
        nj = jnp.sum(jnp.where(iota_ll == j, nxt, 0), axis=0, keepdims=True)

        a_T = jnp.swapaxes(a.astype(jnp.float32), 1, 2)
        pp_T = jnp.swapaxes(pp.astype(jnp.float32), 1, 2)
        nj_T = jnp.swapaxes(nj.astype(jnp.float32), 1, 2)

        m_a = iota_lf == a_T
        f_a = m_a.astype(jnp.float32)
        f_pp = (iota_lf == pp_T).astype(jnp.float32)
        f_nj = (iota_lf == nj_T).astype(jnp.float32)

        x = x_ref[...]
        vecm_sel = jnp.sum(f_a[: L - 1] * vm_ref[...],
                           axis=0, keepdims=True)
        row_pp = jnp.sum(f_pp * x, axis=0, keepdims=True)
        row_nj = jnp.sum(f_nj * x, axis=0, keepdims=True)

        x_ref[...] = jnp.where(m_a, vecm_sel, x)
        nxt_ref[...] = jnp.where(oh_a, nj, nxt)
        prv_ref[...] = jnp.where(iota_ll == nj, a, prv)

        lrows = jnp.concatenate([row_pp, vecm_sel], axis=0)
        rrows = jnp.concatenate([vecm_sel, row_nj], axis=0)
        cat2 = jnp.concatenate([lrows, rrows], axis=-1).reshape(2 * B, 2 * D)
        vecm2, recl2, recr2 = _autoencode(cat2, w_in, b_in, w_out, b_out)
        vecm23 = vecm2.reshape(2, B, D)
        kl = (_kldiv3(recl2.reshape(2, B, D), lrows)
              + _kldiv3(recr2.reshape(2, B, D), rrows))
        kl_T = jnp.swapaxes(kl, 1, 2)

        perp = jnp.where(iota_pl == j, NEG, perp)
        perp = jnp.where(iota_pl == pp, kl_T[0:1], perp)
        perp = jnp.where(iota_pl == a,
                         jnp.where(nj < L, kl_T[1:2], NEG), perp)
        perp_ref[...] = perp
        vm = vm_ref[...]
        vm = jnp.where(f_pp[: L - 1] > 0, vecm23[0:1], vm)
        vm = jnp.where(m_a[: L - 1], vecm23[1:2], vm)
        vm_ref[...] = vm
        return tot

    tot = jax.lax.fori_loop(0, L - 1, step,
                            jnp.zeros((1, 1, B), jnp.float32))
    out_ref[...] = jnp.reshape(jnp.sum(tot) / 2.0 / B / L, (1, 1))


def kernel(input, W_in, b_in, W_outl, b_outl, W_outr, b_outr):
    w_out = jnp.concatenate([W_outl, W_outr], axis=1)
    b_out = jnp.concatenate([b_outl, b_outr])[None, :]
    b_in2 = b_in[None, :]
    out = pl.pallas_call(
        _rae_body,
        out_shape=jax.ShapeDtypeStruct((1, 1), jnp.float32),
        scratch_shapes=[
            pltpu.VMEM((L, B, D), jnp.float32),
            pltpu.VMEM((L - 1, B, D), jnp.float32),
            pltpu.VMEM((L, 1, B), jnp.int32),
            pltpu.VMEM((L, 1, B), jnp.int32),
            pltpu.VMEM((L - 1, 1, B), jnp.float32),
        ],
        compiler_params=pltpu.CompilerParams(
            vmem_limit_bytes=100 * 1024 * 1024),
    )(input, W_in, b_in2, w_out, b_out)
    return out.reshape(1)
